# Initial kernel scaffold; baseline (speedup 1.0000x reference)
import jax
import jax.numpy as jnp
from jax.experimental import pallas as pl


def kernel(drug, cell, gene, edge_index, edge_attr, idx_drug, idx_cell, Wd, bd, Wc, bc, Wg, bg, W1, b1, W2, b2, gn1_w, gn1_b, gn1_ms, gn2_w, gn2_b, gn2_ms, Wout, bout):
    raise NotImplementedError("write your pallas kernel here")


# speedup vs baseline: 3.2049x; 3.2049x over previous
import functools

import jax
import jax.numpy as jnp
from jax import lax
from jax.experimental import pallas as pl
from jax.experimental.pallas import tpu as pltpu
from jax.experimental.pallas import tpu_sc as plsc

NN = 10000
NH = NN // 2
H = 128
E = 320000
EPS = 1e-5
NC, NS = 2, 16
EPT = E // NS
CH = 80
NCH = EPT // CH
SLH = 320
NPH = NS * SLH
TRASH = NPH - 1
SLQ = 80
B = 4096
BPW = B // (NC * NS)

_mesh = plsc.VectorSubcoreMesh(core_axis_name="c", subcore_axis_name="s")


def _remap_dst(dstv, lo):
    def remap(j, carry):
        for k in range(CH // 16):
            v = dstv[j, pl.ds(k * 16, 16)]
            owned = jnp.logical_and(v >= lo, v < lo + NH)
            dstv[j, pl.ds(k * 16, 16)] = jnp.where(owned, v - lo, TRASH)
        return carry
    lax.fori_loop(0, NCH, remap, 0)


@functools.partial(
    pl.kernel,
    out_type=jax.ShapeDtypeStruct((NC, NPH, H), jnp.float32),
    mesh=_mesh,
    scratch_types=[
        pltpu.VMEM((NCH, CH), jnp.int32),
        pltpu.VMEM((NCH, CH), jnp.int32),
        pltpu.VMEM((CH, H), jnp.float32),
        pltpu.VMEM((SLQ, H), jnp.float32),
        pltpu.VMEM_SHARED((NPH, H), jnp.float32),
        pltpu.SemaphoreType.DMA,
    ],
)
def _agg_kernel(x_hbm, src_hbm, dst_hbm, z2_hbm, rows_out,
                srcv, dstv, rows, zb2, acc, sem):
    cid = lax.axis_index("c")
    sid = lax.axis_index("s")

    pltpu.sync_copy(src_hbm.at[sid], srcv)
    pltpu.sync_copy(dst_hbm.at[sid], dstv)

    pltpu.sync_copy(z2_hbm, zb2)
    for q in range(SLH // SLQ):
        pltpu.sync_copy(zb2, acc.at[pl.ds(sid * SLH + q * SLQ, SLQ)])

    _remap_dst(dstv, cid * NH)
    plsc.subcore_barrier()

    def step(j, carry):
        pltpu.async_copy(x_hbm.at[srcv.at[j]], rows, sem).wait()
        pltpu.sync_copy(rows, acc.at[dstv.at[j]], add=True)
        return carry

    lax.fori_loop(0, NCH, step, 0)
    plsc.subcore_barrier()

    for q in range(SLH // SLQ):
        pltpu.sync_copy(acc.at[pl.ds(sid * SLH + q * SLQ, SLQ)], zb2)
        pltpu.sync_copy(zb2, rows_out.at[cid, pl.ds(sid * SLH + q * SLQ, SLQ)])


@functools.partial(
    pl.kernel,
    out_type=(jax.ShapeDtypeStruct((NC * NPH,), jnp.float32),
              jax.ShapeDtypeStruct((NC * NPH,), jnp.float32)),
    mesh=_mesh,
    scratch_types=[
        pltpu.VMEM((NCH, CH), jnp.int32),
        pltpu.VMEM((NCH, CH), jnp.float32),
        pltpu.VMEM((CH,), jnp.float32),
        pltpu.VMEM((SLH,), jnp.float32),
        pltpu.VMEM_SHARED((NPH,), jnp.float32),
        pltpu.VMEM_SHARED((NPH,), jnp.float32),
    ],
)
def _aux_kernel(dst_hbm, ea_hbm, z1_hbm, ones_hbm, e_out, c_out,
                dstv, eav, onesv, zb1, acc_e, acc_c):
    cid = lax.axis_index("c")
    sid = lax.axis_index("s")

    pltpu.sync_copy(dst_hbm.at[sid], dstv)
    pltpu.sync_copy(ea_hbm.at[sid], eav)
    pltpu.sync_copy(ones_hbm, onesv)

    pltpu.sync_copy(z1_hbm, zb1)
    pltpu.sync_copy(zb1, acc_e.at[pl.ds(sid * SLH, SLH)])
    pltpu.sync_copy(zb1, acc_c.at[pl.ds(sid * SLH, SLH)])

    _remap_dst(dstv, cid * NH)
    plsc.subcore_barrier()

    def step(j, carry):
        pltpu.sync_copy(eav.at[j], acc_e.at[dstv.at[j]], add=True)
        pltpu.sync_copy(onesv, acc_c.at[dstv.at[j]], add=True)
        return carry

    lax.fori_loop(0, NCH, step, 0)
    plsc.subcore_barrier()

    pltpu.sync_copy(acc_e.at[pl.ds(sid * SLH, SLH)], zb1)
    pltpu.sync_copy(zb1, e_out.at[pl.ds(cid * NPH + sid * SLH, SLH)])
    pltpu.sync_copy(acc_c.at[pl.ds(sid * SLH, SLH)], zb1)
    pltpu.sync_copy(zb1, c_out.at[pl.ds(cid * NPH + sid * SLH, SLH)])


@functools.partial(
    pl.kernel,
    out_type=jax.ShapeDtypeStruct((B,), jnp.float32),
    mesh=_mesh,
    scratch_types=[
        pltpu.VMEM((BPW,), jnp.int32),
        pltpu.VMEM((BPW,), jnp.int32),
        pltpu.VMEM((BPW,), jnp.float32),
        pltpu.VMEM((BPW,), jnp.float32),
        pltpu.VMEM((BPW,), jnp.float32),
        pltpu.SemaphoreType.DMA,
    ],
)
def _final_kernel(y_hbm, z_hbm, idxd_hbm, idxc_hbm, out_hbm,
                  idv, icv, gy, gz, ov, sem):
    cid = lax.axis_index("c")
    sid = lax.axis_index("s")
    wid = sid * NC + cid
    pltpu.sync_copy(idxd_hbm.at[pl.ds(wid * BPW, BPW)], idv)
    pltpu.sync_copy(idxc_hbm.at[pl.ds(wid * BPW, BPW)], icv)
    pltpu.async_copy(y_hbm.at[idv], gy, sem).wait()
    pltpu.async_copy(z_hbm.at[icv], gz, sem).wait()
    for i in range(BPW // 16):
        s = pl.ds(i * 16, 16)
        ov[s] = gy[s] + gz[s]
    pltpu.sync_copy(ov, out_hbm.at[pl.ds(wid * BPW, BPW)])


def _proj_small(A, W, b):
    M, K = A.shape
    def body(a_ref, w_ref, b_ref, o_ref):
        o_ref[...] = jnp.dot(a_ref[...], w_ref[...],
                             preferred_element_type=jnp.float32) + b_ref[...]
    return pl.pallas_call(
        body,
        out_shape=jax.ShapeDtypeStruct((M, H), jnp.float32),
    )(A, W, b[None])


def _proj_gene(A, W, b, bm=400):
    M, K = A.shape
    def body(a_ref, w_ref, b_ref, o_ref):
        o_ref[...] = jnp.dot(a_ref[...], w_ref[...],
                             preferred_element_type=jnp.float32) + b_ref[...]
    return pl.pallas_call(
        body,
        grid=(M // bm,),
        in_specs=[pl.BlockSpec((bm, K), lambda i: (i, 0)),
                  pl.BlockSpec((K, H), lambda i: (0, 0)),
                  pl.BlockSpec((1, H), lambda i: (0, 0))],
        out_specs=pl.BlockSpec((bm, H), lambda i: (i, 0)),
        out_shape=jax.ShapeDtypeStruct((M, H), jnp.float32),
    )(A, W, b[None])


def _gnorm(x, w, bb, ms):
    mean = jnp.mean(x, axis=0, keepdims=True)
    out = x - mean * ms
    var = jnp.mean(out * out, axis=0, keepdims=True)
    return w * out / jnp.sqrt(var + EPS) + bb


def _mix(a_ref, e_ref, c_ref, w_ref, wr_ref, b_ref):
    agg = jnp.concatenate([a_ref[0, :NH], a_ref[1, :NH]], axis=0)
    x = jnp.dot(agg, w_ref[...], preferred_element_type=jnp.float32)
    es = jnp.concatenate([e_ref[0, :NH], e_ref[1, :NH]], axis=0)
    cs = jnp.concatenate([c_ref[0, :NH], c_ref[1, :NH]], axis=0)
    return x + es * wr_ref[...] + cs * b_ref[...]


def _combine1(aggp, ep, cp, Wm, wr, bb, gw, gb, gms):
    def body(a_ref, e_ref, c_ref, w_ref, wr_ref, b_ref, gw_ref, gb_ref,
             gms_ref, o_ref):
        x = _mix(a_ref, e_ref, c_ref, w_ref, wr_ref, b_ref)
        o_ref[...] = jax.nn.relu(_gnorm(x, gw_ref[...], gb_ref[...],
                                        gms_ref[...]))
    return pl.pallas_call(
        body,
        grid=(1,),
        in_specs=[pl.BlockSpec((NC, NPH, H), lambda i: (0, 0, 0)),
                  pl.BlockSpec((NC, NPH, 1), lambda i: (0, 0, 0)),
                  pl.BlockSpec((NC, NPH, 1), lambda i: (0, 0, 0)),
                  pl.BlockSpec((H, H), lambda i: (0, 0)),
                  pl.BlockSpec((1, H), lambda i: (0, 0)),
                  pl.BlockSpec((1, H), lambda i: (0, 0)),
                  pl.BlockSpec((1, H), lambda i: (0, 0)),
                  pl.BlockSpec((1, H), lambda i: (0, 0)),
                  pl.BlockSpec((1, H), lambda i: (0, 0))],
        out_specs=pl.BlockSpec((NN, H), lambda i: (0, 0)),
        out_shape=jax.ShapeDtypeStruct((NN, H), jnp.float32),
    )(aggp, ep, cp, Wm, wr, bb, gw, gb, gms)


def _combine2(aggp, ep, cp, Wm, wr, bb, gw, gb, gms, wcat, bcat):
    def body(a_ref, e_ref, c_ref, w_ref, wr_ref, b_ref, gw_ref, gb_ref,
             gms_ref, wc_ref, bc_ref, o_ref):
        x = _mix(a_ref, e_ref, c_ref, w_ref, wr_ref, b_ref)
        x = jax.nn.relu(_gnorm(x, gw_ref[...], gb_ref[...], gms_ref[...]))
        o_ref[...] = jnp.dot(x, wc_ref[...],
                             preferred_element_type=jnp.float32) + bc_ref[...]
    return pl.pallas_call(
        body,
        grid=(1,),
        in_specs=[pl.BlockSpec((NC, NPH, H), lambda i: (0, 0, 0)),
                  pl.BlockSpec((NC, NPH, 1), lambda i: (0, 0, 0)),
                  pl.BlockSpec((NC, NPH, 1), lambda i: (0, 0, 0)),
                  pl.BlockSpec((H, H), lambda i: (0, 0)),
                  pl.BlockSpec((1, H), lambda i: (0, 0)),
                  pl.BlockSpec((1, H), lambda i: (0, 0)),
                  pl.BlockSpec((1, H), lambda i: (0, 0)),
                  pl.BlockSpec((1, H), lambda i: (0, 0)),
                  pl.BlockSpec((1, H), lambda i: (0, 0)),
                  pl.BlockSpec((H, 2), lambda i: (0, 0)),
                  pl.BlockSpec((1, 2), lambda i: (0, 0))],
        out_specs=pl.BlockSpec((NN, 2), lambda i: (0, 0)),
        out_shape=jax.ShapeDtypeStruct((NN, 2), jnp.float32),
    )(aggp, ep, cp, Wm, wr, bb, gw, gb, gms, wcat, bcat)


def kernel(drug, cell, gene, edge_index, edge_attr, idx_drug, idx_cell,
           Wd, bd, Wc, bc, Wg, bg, W1, b1, W2, b2,
           gn1_w, gn1_b, gn1_ms, gn2_w, gn2_b, gn2_ms, Wout, bout):
    src3 = edge_index[0].reshape(NS, NCH, CH)
    dst3 = edge_index[1].reshape(NS, NCH, CH)
    ea3 = edge_attr.reshape(NS, NCH, CH).astype(jnp.float32)
    z2 = jnp.zeros((SLQ, H), jnp.float32)
    z1 = jnp.zeros((SLH,), jnp.float32)
    ones = jnp.ones((CH,), jnp.float32)

    x0 = jnp.concatenate([_proj_small(drug, Wd, bd),
                          _proj_small(cell, Wc, bc),
                          _proj_gene(gene, Wg, bg)], axis=0)

    ep, cp = _aux_kernel(dst3, ea3, z1, ones)
    ep3 = ep.reshape(NC, NPH, 1)
    cp3 = cp.reshape(NC, NPH, 1)

    aggp = _agg_kernel(x0, src3, dst3, z2)
    x1 = _combine1(aggp, ep3, cp3, W1[:H], W1[H:H + 1], b1[None],
                   gn1_w[None], gn1_b[None], gn1_ms[None])

    aggp2 = _agg_kernel(x1, src3, dst3, z2)

    wcat = jnp.concatenate([Wout[:H], Wout[H:]], axis=1)
    bcat = jnp.concatenate([bout, jnp.zeros((1,), jnp.float32)])[None]
    yz = _combine2(aggp2, ep3, cp3, W2[:H], W2[H:H + 1], b2[None],
                   gn2_w[None], gn2_b[None], gn2_ms[None], wcat, bcat)

    out = _final_kernel(yz[:, 0], yz[:, 1], idx_drug, idx_cell)
    return out[:, None]


# speedup vs baseline: 7.2155x; 2.2514x over previous
import functools

import jax
import jax.numpy as jnp
from jax import lax
from jax.experimental import pallas as pl
from jax.experimental.pallas import tpu as pltpu
from jax.experimental.pallas import tpu_sc as plsc

NN = 10000
H = 128
E = 320000
EPS = 1e-5
NC, NS = 2, 16
NW = NC * NS
EPW = E // NW
CH = 80
NCH = EPW // CH
SLF = 640
NPF = NS * SLF
B = 4096
BPW = B // NW

_mesh = plsc.VectorSubcoreMesh(core_axis_name="c", subcore_axis_name="s")


@functools.partial(
    pl.kernel,
    out_type=jax.ShapeDtypeStruct((NC, NPF, H), jnp.float32),
    mesh=_mesh,
    scratch_types=[
        pltpu.VMEM((NCH, CH), jnp.int32),
        pltpu.VMEM((2, CH), jnp.int32),
        pltpu.VMEM((2, CH), jnp.int32),
        pltpu.VMEM((2, CH, H), jnp.float32),
        pltpu.VMEM_SHARED((NPF, H), jnp.float32),
        pltpu.SemaphoreType.DMA,
        pltpu.SemaphoreType.DMA,
    ],
)
def _agg_kernel(x_hbm, pk_hbm, z2_hbm, rows_out,
                pk, srcb, dstb, rows, acc, sem0, sem1):
    cid = lax.axis_index("c")
    sid = lax.axis_index("s")
    wid = sid * NC + cid
    sems = (sem0, sem1)

    pltpu.sync_copy(pk_hbm.at[wid], pk)

    pltpu.sync_copy(z2_hbm, rows.at[0])
    for q in range(SLF // CH):
        pltpu.sync_copy(rows.at[0], acc.at[pl.ds(sid * SLF + q * CH, CH)])
    plsc.subcore_barrier()

    def unpack(c, s):
        for k in range(CH // 16):
            v = pk[c, pl.ds(k * 16, 16)]
            srcb[s, pl.ds(k * 16, 16)] = jnp.bitwise_and(v, 16383)
            dstb[s, pl.ds(k * 16, 16)] = jnp.right_shift(v, 14)

    def gather_start(c, s):
        pltpu.async_copy(x_hbm.at[srcb.at[s]], rows.at[s], sems[s])

    def gather_wait(s):
        pltpu.make_async_copy(x_hbm.at[srcb.at[s]], rows.at[s],
                              sems[s]).wait()

    def scatter(s):
        pltpu.sync_copy(rows.at[s], acc.at[dstb.at[s]], add=True)

    unpack(0, 0)
    gather_start(0, 0)

    def pair(i, carry):
        for b in range(2):
            c = 2 * i + b

            @pl.when(c + 1 < NCH)
            def _():
                unpack(c + 1, 1 - b)

            gather_wait(b)

            @pl.when(c + 1 < NCH)
            def _():
                gather_start(c + 1, 1 - b)

            scatter(b)
        return carry

    lax.fori_loop(0, NCH // 2, pair, 0)
    if NCH % 2:
        gather_wait(0)
        scatter(0)
    plsc.subcore_barrier()

    for q in range(SLF // CH):
        pltpu.sync_copy(acc.at[pl.ds(sid * SLF + q * CH, CH)], rows.at[0])
        pltpu.sync_copy(rows.at[0],
                        rows_out.at[cid, pl.ds(sid * SLF + q * CH, CH)])


@functools.partial(
    pl.kernel,
    out_type=(jax.ShapeDtypeStruct((NC * NPF,), jnp.float32),
              jax.ShapeDtypeStruct((NC * NPF,), jnp.float32)),
    mesh=_mesh,
    scratch_types=[
        pltpu.VMEM((NCH, CH), jnp.int32),
        pltpu.VMEM((NCH, CH), jnp.float32),
        pltpu.VMEM((CH,), jnp.float32),
        pltpu.VMEM((SLF,), jnp.float32),
        pltpu.VMEM_SHARED((NPF,), jnp.float32),
        pltpu.VMEM_SHARED((NPF,), jnp.float32),
    ],
)
def _aux_kernel(dst_hbm, ea_hbm, z1_hbm, ones_hbm, e_out, c_out,
                dstv, eav, onesv, zb1, acc_e, acc_c):
    cid = lax.axis_index("c")
    sid = lax.axis_index("s")
    wid = sid * NC + cid

    pltpu.sync_copy(dst_hbm.at[wid], dstv)
    pltpu.sync_copy(ea_hbm.at[wid], eav)
    pltpu.sync_copy(ones_hbm, onesv)

    pltpu.sync_copy(z1_hbm, zb1)
    pltpu.sync_copy(zb1, acc_e.at[pl.ds(sid * SLF, SLF)])
    pltpu.sync_copy(zb1, acc_c.at[pl.ds(sid * SLF, SLF)])
    plsc.subcore_barrier()

    def step(j, carry):
        pltpu.sync_copy(eav.at[j], acc_e.at[dstv.at[j]], add=True)
        pltpu.sync_copy(onesv, acc_c.at[dstv.at[j]], add=True)
        return carry

    lax.fori_loop(0, NCH, step, 0)
    plsc.subcore_barrier()

    pltpu.sync_copy(acc_e.at[pl.ds(sid * SLF, SLF)], zb1)
    pltpu.sync_copy(zb1, e_out.at[pl.ds(cid * NPF + sid * SLF, SLF)])
    pltpu.sync_copy(acc_c.at[pl.ds(sid * SLF, SLF)], zb1)
    pltpu.sync_copy(zb1, c_out.at[pl.ds(cid * NPF + sid * SLF, SLF)])


@functools.partial(
    pl.kernel,
    out_type=jax.ShapeDtypeStruct((B,), jnp.float32),
    mesh=_mesh,
    scratch_types=[
        pltpu.VMEM((BPW,), jnp.int32),
        pltpu.VMEM((BPW,), jnp.int32),
        pltpu.VMEM((BPW,), jnp.float32),
        pltpu.VMEM((BPW,), jnp.float32),
        pltpu.VMEM((BPW,), jnp.float32),
        pltpu.SemaphoreType.DMA,
    ],
)
def _final_kernel(y_hbm, z_hbm, idxd_hbm, idxc_hbm, out_hbm,
                  idv, icv, gy, gz, ov, sem):
    cid = lax.axis_index("c")
    sid = lax.axis_index("s")
    wid = sid * NC + cid
    pltpu.sync_copy(idxd_hbm.at[pl.ds(wid * BPW, BPW)], idv)
    pltpu.sync_copy(idxc_hbm.at[pl.ds(wid * BPW, BPW)], icv)
    pltpu.async_copy(y_hbm.at[idv], gy, sem).wait()
    pltpu.async_copy(z_hbm.at[icv], gz, sem).wait()
    for i in range(BPW // 16):
        s = pl.ds(i * 16, 16)
        ov[s] = gy[s] + gz[s]
    pltpu.sync_copy(ov, out_hbm.at[pl.ds(wid * BPW, BPW)])


def _proj_small(A, W, b):
    M, K = A.shape
    def body(a_ref, w_ref, b_ref, o_ref):
        o_ref[...] = jnp.dot(a_ref[...], w_ref[...],
                             preferred_element_type=jnp.float32) + b_ref[...]
    return pl.pallas_call(
        body,
        out_shape=jax.ShapeDtypeStruct((M, H), jnp.float32),
    )(A, W, b[None])


def _proj_gene(A, W, b, bm=400):
    M, K = A.shape
    def body(a_ref, w_ref, b_ref, o_ref):
        o_ref[...] = jnp.dot(a_ref[...], w_ref[...],
                             preferred_element_type=jnp.float32) + b_ref[...]
    return pl.pallas_call(
        body,
        grid=(M // bm,),
        in_specs=[pl.BlockSpec((bm, K), lambda i: (i, 0)),
                  pl.BlockSpec((K, H), lambda i: (0, 0)),
                  pl.BlockSpec((1, H), lambda i: (0, 0))],
        out_specs=pl.BlockSpec((bm, H), lambda i: (i, 0)),
        out_shape=jax.ShapeDtypeStruct((M, H), jnp.float32),
    )(A, W, b[None])


def _gnorm(x, w, bb, ms):
    mean = jnp.mean(x, axis=0, keepdims=True)
    out = x - mean * ms
    var = jnp.mean(out * out, axis=0, keepdims=True)
    return w * out / jnp.sqrt(var + EPS) + bb


def _mix(a_ref, e_ref, c_ref, w_ref, wr_ref, b_ref):
    agg = a_ref[0] + a_ref[1]
    x = jnp.dot(agg, w_ref[...], preferred_element_type=jnp.float32)
    es = e_ref[0] + e_ref[1]
    cs = c_ref[0] + c_ref[1]
    return x + es * wr_ref[...] + cs * b_ref[...]


def _combine1(aggp, ep, cp, Wm, wr, bb, gw, gb, gms):
    def body(a_ref, e_ref, c_ref, w_ref, wr_ref, b_ref, gw_ref, gb_ref,
             gms_ref, o_ref):
        x = _mix(a_ref, e_ref, c_ref, w_ref, wr_ref, b_ref)
        o_ref[...] = jax.nn.relu(_gnorm(x, gw_ref[...], gb_ref[...],
                                        gms_ref[...]))
    return pl.pallas_call(
        body,
        grid=(1,),
        in_specs=[pl.BlockSpec((NC, NN, H), lambda i: (0, 0, 0)),
                  pl.BlockSpec((NC, NN, 1), lambda i: (0, 0, 0)),
                  pl.BlockSpec((NC, NN, 1), lambda i: (0, 0, 0)),
                  pl.BlockSpec((H, H), lambda i: (0, 0)),
                  pl.BlockSpec((1, H), lambda i: (0, 0)),
                  pl.BlockSpec((1, H), lambda i: (0, 0)),
                  pl.BlockSpec((1, H), lambda i: (0, 0)),
                  pl.BlockSpec((1, H), lambda i: (0, 0)),
                  pl.BlockSpec((1, H), lambda i: (0, 0))],
        out_specs=pl.BlockSpec((NN, H), lambda i: (0, 0)),
        out_shape=jax.ShapeDtypeStruct((NN, H), jnp.float32),
    )(aggp, ep, cp, Wm, wr, bb, gw, gb, gms)


def _combine2(aggp, ep, cp, Wm, wr, bb, gw, gb, gms, wcat, bcat):
    def body(a_ref, e_ref, c_ref, w_ref, wr_ref, b_ref, gw_ref, gb_ref,
             gms_ref, wc_ref, bc_ref, o_ref):
        x = _mix(a_ref, e_ref, c_ref, w_ref, wr_ref, b_ref)
        x = jax.nn.relu(_gnorm(x, gw_ref[...], gb_ref[...], gms_ref[...]))
        o_ref[...] = jnp.dot(x, wc_ref[...],
                             preferred_element_type=jnp.float32) + bc_ref[...]
    return pl.pallas_call(
        body,
        grid=(1,),
        in_specs=[pl.BlockSpec((NC, NN, H), lambda i: (0, 0, 0)),
                  pl.BlockSpec((NC, NN, 1), lambda i: (0, 0, 0)),
                  pl.BlockSpec((NC, NN, 1), lambda i: (0, 0, 0)),
                  pl.BlockSpec((H, H), lambda i: (0, 0)),
                  pl.BlockSpec((1, H), lambda i: (0, 0)),
                  pl.BlockSpec((1, H), lambda i: (0, 0)),
                  pl.BlockSpec((1, H), lambda i: (0, 0)),
                  pl.BlockSpec((1, H), lambda i: (0, 0)),
                  pl.BlockSpec((1, H), lambda i: (0, 0)),
                  pl.BlockSpec((H, 2), lambda i: (0, 0)),
                  pl.BlockSpec((1, 2), lambda i: (0, 0))],
        out_specs=pl.BlockSpec((NN, 2), lambda i: (0, 0)),
        out_shape=jax.ShapeDtypeStruct((NN, 2), jnp.float32),
    )(aggp, ep, cp, Wm, wr, bb, gw, gb, gms, wcat, bcat)


def kernel(drug, cell, gene, edge_index, edge_attr, idx_drug, idx_cell,
           Wd, bd, Wc, bc, Wg, bg, W1, b1, W2, b2,
           gn1_w, gn1_b, gn1_ms, gn2_w, gn2_b, gn2_ms, Wout, bout):
    pk3 = (edge_index[0] + (edge_index[1] << 14)).reshape(NW, NCH, CH)
    dst3 = edge_index[1].reshape(NW, NCH, CH)
    ea3 = edge_attr.reshape(NW, NCH, CH).astype(jnp.float32)
    z2 = jnp.zeros((CH, H), jnp.float32)
    z1 = jnp.zeros((SLF,), jnp.float32)
    ones = jnp.ones((CH,), jnp.float32)

    x0 = jnp.concatenate([_proj_small(drug, Wd, bd),
                          _proj_small(cell, Wc, bc),
                          _proj_gene(gene, Wg, bg)], axis=0)

    ep, cp = _aux_kernel(dst3, ea3, z1, ones)
    ep3 = ep.reshape(NC, NPF, 1)
    cp3 = cp.reshape(NC, NPF, 1)

    aggp = _agg_kernel(x0, pk3, z2)
    x1 = _combine1(aggp, ep3, cp3, W1[:H], W1[H:H + 1], b1[None],
                   gn1_w[None], gn1_b[None], gn1_ms[None])

    aggp2 = _agg_kernel(x1, pk3, z2)

    wcat = jnp.concatenate([Wout[:H], Wout[H:]], axis=1)
    bcat = jnp.concatenate([bout, jnp.zeros((1,), jnp.float32)])[None]
    yz = _combine2(aggp2, ep3, cp3, W2[:H], W2[H:H + 1], b2[None],
                   gn2_w[None], gn2_b[None], gn2_ms[None], wcat, bcat)

    out = _final_kernel(yz[:, 0], yz[:, 1], idx_drug, idx_cell)
    return out[:, None]


# speedup vs baseline: 7.2484x; 1.0046x over previous
import functools

import jax
import jax.numpy as jnp
from jax import lax
from jax.experimental import pallas as pl
from jax.experimental.pallas import tpu as pltpu
from jax.experimental.pallas import tpu_sc as plsc

NN = 10000
H = 128
E = 320000
EPS = 1e-5
NC, NS = 2, 16
NW = NC * NS
EPW = E // NW
CH = 80
NCH = EPW // CH
SLF = 640
NPF = NS * SLF
B = 4096
BPW = B // NW

_mesh = plsc.VectorSubcoreMesh(core_axis_name="c", subcore_axis_name="s")


@functools.partial(
    pl.kernel,
    out_type=jax.ShapeDtypeStruct((NC, NPF, H), jnp.float32),
    mesh=_mesh,
    scratch_types=[
        pltpu.VMEM((NCH, CH), jnp.int32),
        pltpu.VMEM((2, CH), jnp.int32),
        pltpu.VMEM((2, CH), jnp.int32),
        pltpu.VMEM((2, CH, H), jnp.float32),
        pltpu.VMEM_SHARED((NPF, H), jnp.float32),
        pltpu.SemaphoreType.DMA,
        pltpu.SemaphoreType.DMA,
    ],
)
def _agg_kernel(x_hbm, pk_hbm, z2_hbm, rows_out,
                pk, srcb, dstb, rows, acc, sem0, sem1):
    cid = lax.axis_index("c")
    sid = lax.axis_index("s")
    wid = sid * NC + cid
    sems = (sem0, sem1)

    pltpu.sync_copy(pk_hbm.at[wid], pk)

    pltpu.sync_copy(z2_hbm, rows.at[0])
    for q in range(SLF // CH):
        pltpu.sync_copy(rows.at[0], acc.at[pl.ds(sid * SLF + q * CH, CH)])
    plsc.subcore_barrier()

    def unpack(c, s):
        for k in range(CH // 16):
            v = pk[c, pl.ds(k * 16, 16)]
            srcb[s, pl.ds(k * 16, 16)] = jnp.bitwise_and(v, 16383)
            dstb[s, pl.ds(k * 16, 16)] = jnp.right_shift(v, 14)

    def gather_start(c, s):
        pltpu.async_copy(x_hbm.at[srcb.at[s]], rows.at[s], sems[s])

    def gather_wait(s):
        pltpu.make_async_copy(x_hbm.at[srcb.at[s]], rows.at[s],
                              sems[s]).wait()

    def scatter(s):
        pltpu.sync_copy(rows.at[s], acc.at[dstb.at[s]], add=True)

    unpack(0, 0)
    gather_start(0, 0)

    def pair(i, carry):
        for b in range(2):
            c = 2 * i + b
            unpack(c + 1, 1 - b)
            gather_wait(b)
            gather_start(c + 1, 1 - b)
            scatter(b)
        return carry

    lax.fori_loop(0, (NCH - 1) // 2, pair, 0)
    gather_wait((NCH - 1) % 2)
    scatter((NCH - 1) % 2)
    plsc.subcore_barrier()

    for q in range(SLF // CH):
        pltpu.sync_copy(acc.at[pl.ds(sid * SLF + q * CH, CH)], rows.at[0])
        pltpu.sync_copy(rows.at[0],
                        rows_out.at[cid, pl.ds(sid * SLF + q * CH, CH)])


@functools.partial(
    pl.kernel,
    out_type=(jax.ShapeDtypeStruct((NC * NPF,), jnp.float32),
              jax.ShapeDtypeStruct((NC * NPF,), jnp.float32)),
    mesh=_mesh,
    scratch_types=[
        pltpu.VMEM((NCH, CH), jnp.int32),
        pltpu.VMEM((NCH, CH), jnp.float32),
        pltpu.VMEM((CH,), jnp.float32),
        pltpu.VMEM((SLF,), jnp.float32),
        pltpu.VMEM_SHARED((NPF,), jnp.float32),
        pltpu.VMEM_SHARED((NPF,), jnp.float32),
    ],
)
def _aux_kernel(dst_hbm, ea_hbm, z1_hbm, ones_hbm, e_out, c_out,
                dstv, eav, onesv, zb1, acc_e, acc_c):
    cid = lax.axis_index("c")
    sid = lax.axis_index("s")
    wid = sid * NC + cid

    pltpu.sync_copy(dst_hbm.at[wid], dstv)
    pltpu.sync_copy(ea_hbm.at[wid], eav)
    pltpu.sync_copy(ones_hbm, onesv)

    pltpu.sync_copy(z1_hbm, zb1)
    pltpu.sync_copy(zb1, acc_e.at[pl.ds(sid * SLF, SLF)])
    pltpu.sync_copy(zb1, acc_c.at[pl.ds(sid * SLF, SLF)])
    plsc.subcore_barrier()

    def step(j, carry):
        pltpu.sync_copy(eav.at[j], acc_e.at[dstv.at[j]], add=True)
        pltpu.sync_copy(onesv, acc_c.at[dstv.at[j]], add=True)
        return carry

    lax.fori_loop(0, NCH, step, 0)
    plsc.subcore_barrier()

    pltpu.sync_copy(acc_e.at[pl.ds(sid * SLF, SLF)], zb1)
    pltpu.sync_copy(zb1, e_out.at[pl.ds(cid * NPF + sid * SLF, SLF)])
    pltpu.sync_copy(acc_c.at[pl.ds(sid * SLF, SLF)], zb1)
    pltpu.sync_copy(zb1, c_out.at[pl.ds(cid * NPF + sid * SLF, SLF)])


@functools.partial(
    pl.kernel,
    out_type=jax.ShapeDtypeStruct((B,), jnp.float32),
    mesh=_mesh,
    scratch_types=[
        pltpu.VMEM((BPW,), jnp.int32),
        pltpu.VMEM((BPW,), jnp.int32),
        pltpu.VMEM((BPW,), jnp.float32),
        pltpu.VMEM((BPW,), jnp.float32),
        pltpu.VMEM((BPW,), jnp.float32),
        pltpu.SemaphoreType.DMA,
    ],
)
def _final_kernel(y_hbm, z_hbm, idxd_hbm, idxc_hbm, out_hbm,
                  idv, icv, gy, gz, ov, sem):
    cid = lax.axis_index("c")
    sid = lax.axis_index("s")
    wid = sid * NC + cid
    pltpu.sync_copy(idxd_hbm.at[pl.ds(wid * BPW, BPW)], idv)
    pltpu.sync_copy(idxc_hbm.at[pl.ds(wid * BPW, BPW)], icv)
    pltpu.async_copy(y_hbm.at[idv], gy, sem).wait()
    pltpu.async_copy(z_hbm.at[icv], gz, sem).wait()
    for i in range(BPW // 16):
        s = pl.ds(i * 16, 16)
        ov[s] = gy[s] + gz[s]
    pltpu.sync_copy(ov, out_hbm.at[pl.ds(wid * BPW, BPW)])


def _proj_small(A, W, b):
    M, K = A.shape
    def body(a_ref, w_ref, b_ref, o_ref):
        o_ref[...] = jnp.dot(a_ref[...], w_ref[...],
                             preferred_element_type=jnp.float32) + b_ref[...]
    return pl.pallas_call(
        body,
        out_shape=jax.ShapeDtypeStruct((M, H), jnp.float32),
    )(A, W, b[None])


def _proj_gene(A, W, b, bm=400):
    M, K = A.shape
    def body(a_ref, w_ref, b_ref, o_ref):
        o_ref[...] = jnp.dot(a_ref[...], w_ref[...],
                             preferred_element_type=jnp.float32) + b_ref[...]
    return pl.pallas_call(
        body,
        grid=(M // bm,),
        in_specs=[pl.BlockSpec((bm, K), lambda i: (i, 0)),
                  pl.BlockSpec((K, H), lambda i: (0, 0)),
                  pl.BlockSpec((1, H), lambda i: (0, 0))],
        out_specs=pl.BlockSpec((bm, H), lambda i: (i, 0)),
        out_shape=jax.ShapeDtypeStruct((M, H), jnp.float32),
    )(A, W, b[None])


def _gnorm(x, w, bb, ms):
    mean = jnp.mean(x, axis=0, keepdims=True)
    out = x - mean * ms
    var = jnp.mean(out * out, axis=0, keepdims=True)
    return w * out / jnp.sqrt(var + EPS) + bb


def _mix(a_ref, e_ref, c_ref, w_ref, wr_ref, b_ref):
    agg = a_ref[0] + a_ref[1]
    x = jnp.dot(agg, w_ref[...], preferred_element_type=jnp.float32)
    es = e_ref[0] + e_ref[1]
    cs = c_ref[0] + c_ref[1]
    return x + es * wr_ref[...] + cs * b_ref[...]


def _combine1(aggp, ep, cp, Wm, wr, bb, gw, gb, gms):
    def body(a_ref, e_ref, c_ref, w_ref, wr_ref, b_ref, gw_ref, gb_ref,
             gms_ref, o_ref):
        x = _mix(a_ref, e_ref, c_ref, w_ref, wr_ref, b_ref)
        o_ref[...] = jax.nn.relu(_gnorm(x, gw_ref[...], gb_ref[...],
                                        gms_ref[...]))
    return pl.pallas_call(
        body,
        grid=(1,),
        in_specs=[pl.BlockSpec((NC, NN, H), lambda i: (0, 0, 0)),
                  pl.BlockSpec((NC, NN, 1), lambda i: (0, 0, 0)),
                  pl.BlockSpec((NC, NN, 1), lambda i: (0, 0, 0)),
                  pl.BlockSpec((H, H), lambda i: (0, 0)),
                  pl.BlockSpec((1, H), lambda i: (0, 0)),
                  pl.BlockSpec((1, H), lambda i: (0, 0)),
                  pl.BlockSpec((1, H), lambda i: (0, 0)),
                  pl.BlockSpec((1, H), lambda i: (0, 0)),
                  pl.BlockSpec((1, H), lambda i: (0, 0))],
        out_specs=pl.BlockSpec((NN, H), lambda i: (0, 0)),
        out_shape=jax.ShapeDtypeStruct((NN, H), jnp.float32),
    )(aggp, ep, cp, Wm, wr, bb, gw, gb, gms)


def _combine2(aggp, ep, cp, Wm, wr, bb, gw, gb, gms, wcat, bcat):
    def body(a_ref, e_ref, c_ref, w_ref, wr_ref, b_ref, gw_ref, gb_ref,
             gms_ref, wc_ref, bc_ref, o_ref):
        x = _mix(a_ref, e_ref, c_ref, w_ref, wr_ref, b_ref)
        x = jax.nn.relu(_gnorm(x, gw_ref[...], gb_ref[...], gms_ref[...]))
        o_ref[...] = jnp.dot(x, wc_ref[...],
                             preferred_element_type=jnp.float32) + bc_ref[...]
    return pl.pallas_call(
        body,
        grid=(1,),
        in_specs=[pl.BlockSpec((NC, NN, H), lambda i: (0, 0, 0)),
                  pl.BlockSpec((NC, NN, 1), lambda i: (0, 0, 0)),
                  pl.BlockSpec((NC, NN, 1), lambda i: (0, 0, 0)),
                  pl.BlockSpec((H, H), lambda i: (0, 0)),
                  pl.BlockSpec((1, H), lambda i: (0, 0)),
                  pl.BlockSpec((1, H), lambda i: (0, 0)),
                  pl.BlockSpec((1, H), lambda i: (0, 0)),
                  pl.BlockSpec((1, H), lambda i: (0, 0)),
                  pl.BlockSpec((1, H), lambda i: (0, 0)),
                  pl.BlockSpec((H, 2), lambda i: (0, 0)),
                  pl.BlockSpec((1, 2), lambda i: (0, 0))],
        out_specs=pl.BlockSpec((NN, 2), lambda i: (0, 0)),
        out_shape=jax.ShapeDtypeStruct((NN, 2), jnp.float32),
    )(aggp, ep, cp, Wm, wr, bb, gw, gb, gms, wcat, bcat)


def kernel(drug, cell, gene, edge_index, edge_attr, idx_drug, idx_cell,
           Wd, bd, Wc, bc, Wg, bg, W1, b1, W2, b2,
           gn1_w, gn1_b, gn1_ms, gn2_w, gn2_b, gn2_ms, Wout, bout):
    pk3 = (edge_index[0] + (edge_index[1] << 14)).reshape(NW, NCH, CH)
    dst3 = edge_index[1].reshape(NW, NCH, CH)
    ea3 = edge_attr.reshape(NW, NCH, CH).astype(jnp.float32)
    z2 = jnp.zeros((CH, H), jnp.float32)
    z1 = jnp.zeros((SLF,), jnp.float32)
    ones = jnp.ones((CH,), jnp.float32)

    x0 = jnp.concatenate([_proj_small(drug, Wd, bd),
                          _proj_small(cell, Wc, bc),
                          _proj_gene(gene, Wg, bg)], axis=0)

    ep, cp = _aux_kernel(dst3, ea3, z1, ones)
    ep3 = ep.reshape(NC, NPF, 1)
    cp3 = cp.reshape(NC, NPF, 1)

    aggp = _agg_kernel(x0, pk3, z2)
    x1 = _combine1(aggp, ep3, cp3, W1[:H], W1[H:H + 1], b1[None],
                   gn1_w[None], gn1_b[None], gn1_ms[None])

    aggp2 = _agg_kernel(x1, pk3, z2)

    wcat = jnp.concatenate([Wout[:H], Wout[H:]], axis=1)
    bcat = jnp.concatenate([bout, jnp.zeros((1,), jnp.float32)])[None]
    yz = _combine2(aggp2, ep3, cp3, W2[:H], W2[H:H + 1], b2[None],
                   gn2_w[None], gn2_b[None], gn2_ms[None], wcat, bcat)

    out = _final_kernel(yz[:, 0], yz[:, 1], idx_drug, idx_cell)
    return out[:, None]
